# baseline (device time: 81496 ns/iter reference)
import jax
import jax.numpy as jnp
from jax import lax
from jax.experimental import pallas as pl
from jax.experimental.pallas import tpu as pltpu

N_DEV = 8
M_BLK = 512
K_BLK = 512
K_TOT = 4096
N_TOT = 8192
NB = 512
N_STRIPES = N_TOT // NB
S = 4
FP8 = jnp.float8_e4m3fn


def kernel(x, w_mat, scale_x, scale_w):
    def body(x_ref, w_hbm, sx_ref, sw_ref, out_hbm,
             xq, a8, af, wbuf, obuf,
             send_sems, recv_sems, w_sems, out_sems):
        my = lax.axis_index("i")

        def w_dma(t, slot):
            return pltpu.make_async_copy(
                w_hbm.at[:, pl.ds(t * NB, NB)],
                wbuf.at[slot],
                w_sems.at[slot],
            )

        pending = {}
        for t in range(S):
            d = w_dma(t, t)
            d.start()
            pending[t] = d

        xq[...] = x_ref[...].astype(FP8)
        a8[:, pl.ds(my * K_BLK, K_BLK)] = xq[pl.ds(my * M_BLK, M_BLK), :]

        bar = pltpu.get_barrier_semaphore()
        for d in range(1, N_DEV):
            peer = lax.rem(my + d, N_DEV)
            pl.semaphore_signal(bar, inc=1, device_id=(peer,),
                                device_id_type=pl.DeviceIdType.MESH)
        pl.semaphore_wait(bar, N_DEV - 1)

        rdmas = []
        for d in range(1, N_DEV):
            dst = lax.rem(my + d, N_DEV)
            r = pltpu.make_async_remote_copy(
                src_ref=xq.at[pl.ds(dst * M_BLK, M_BLK), :],
                dst_ref=a8.at[:, pl.ds(my * K_BLK, K_BLK)],
                send_sem=send_sems.at[d - 1],
                recv_sem=recv_sems.at[my],
                device_id=(dst,),
                device_id_type=pl.DeviceIdType.MESH,
            )
            r.start()
            rdmas.append(r)

        for d in range(1, N_DEV):
            k = lax.rem(my + d, N_DEV)
            pltpu.make_async_remote_copy(
                src_ref=xq.at[pl.ds(k * M_BLK, M_BLK), :],
                dst_ref=a8.at[:, pl.ds(k * K_BLK, K_BLK)],
                send_sem=send_sems.at[0],
                recv_sem=recv_sems.at[k],
                device_id=(my,),
                device_id_type=pl.DeviceIdType.MESH,
            ).wait_recv()

        s = sx_ref[0] * sw_ref[0]

        af[...] = a8[...].astype(jnp.float32)

        for t in range(N_STRIPES):
            slot = t % S
            oslot = t % 2
            pending[slot].wait()
            prod = lax.dot_general(
                af[...], wbuf[slot], (((1,), (0,)), ((), ())),
                preferred_element_type=jnp.float32,
            )
            if t + S < N_STRIPES:
                nd = w_dma(t + S, slot)
                nd.start()
                pending[slot] = nd
            y = prod * s
            res = y / (1.0 + jnp.exp(-jnp.clip(y, -60.0, 60.0)))
            if t >= 2:
                pltpu.make_async_copy(
                    obuf.at[oslot],
                    out_hbm.at[:, pl.ds((t - 2) * NB, NB)],
                    out_sems.at[oslot],
                ).wait()
            obuf[oslot] = res
            pltpu.make_async_copy(
                obuf.at[oslot],
                out_hbm.at[:, pl.ds(t * NB, NB)],
                out_sems.at[oslot],
            ).start()

        for t in (N_STRIPES - 2, N_STRIPES - 1):
            pltpu.make_async_copy(
                obuf.at[t % 2],
                out_hbm.at[:, pl.ds(t * NB, NB)],
                out_sems.at[t % 2],
            ).wait()
        for r in rdmas:
            r.wait_send()

    return pl.pallas_call(
        body,
        out_shape=jax.ShapeDtypeStruct((M_BLK, N_TOT), jnp.float32),
        in_specs=[
            pl.BlockSpec(memory_space=pltpu.VMEM),
            pl.BlockSpec(memory_space=pl.ANY),
            pl.BlockSpec(memory_space=pltpu.SMEM),
            pl.BlockSpec(memory_space=pltpu.SMEM),
        ],
        out_specs=pl.BlockSpec(memory_space=pl.ANY),
        scratch_shapes=[
            pltpu.VMEM((K_TOT, K_BLK), FP8),
            pltpu.VMEM((M_BLK, K_TOT), FP8),
            pltpu.VMEM((M_BLK, K_TOT), jnp.float32),
            pltpu.VMEM((S, K_TOT, NB), jnp.float32),
            pltpu.VMEM((2, M_BLK, NB), jnp.float32),
            pltpu.SemaphoreType.DMA((N_DEV - 1,)),
            pltpu.SemaphoreType.DMA((N_DEV,)),
            pltpu.SemaphoreType.DMA((S,)),
            pltpu.SemaphoreType.DMA((2,)),
        ],
        compiler_params=pltpu.CompilerParams(
            collective_id=0, vmem_limit_bytes=100 * 1024 * 1024
        ),
    )(x, w_mat, scale_x, scale_w)


# device time: 70681 ns/iter; 1.1530x vs baseline; 1.1530x over previous
import jax
import jax.numpy as jnp
from jax import lax
from jax.experimental import pallas as pl
from jax.experimental.pallas import tpu as pltpu

N_DEV = 8
M_BLK = 512
K_BLK = 512
K_TOT = 4096
N_TOT = 8192
NB = 512
N_STRIPES = N_TOT // NB
S = 5
FP8 = jnp.float8_e4m3fn


def kernel(x, w_mat, scale_x, scale_w):
    def body(x_hbm, w_hbm, sx_ref, sw_ref, out_hbm,
             xv8, xq8, a8, wbuf, obuf,
             x_sems, send_sems, recv_sems, w_sems, out_sems):
        my = lax.axis_index("i")

        def x_dma(d):
            dst = lax.rem(my + d, N_DEV)
            return pltpu.make_async_copy(
                x_hbm.at[pl.ds(dst * M_BLK, M_BLK), :],
                xv8.at[d],
                x_sems.at[d],
            )

        xdmas = []
        for d in range(N_DEV):
            dma = x_dma(d)
            dma.start()
            xdmas.append(dma)

        def w_dma(t, slot):
            return pltpu.make_async_copy(
                w_hbm.at[:, pl.ds(t * NB, NB)],
                wbuf.at[slot],
                w_sems.at[slot],
            )

        pending = {}
        for t in range(S):
            d = w_dma(t, t)
            d.start()
            pending[t] = d

        bar = pltpu.get_barrier_semaphore()
        for d in range(1, N_DEV):
            peer = lax.rem(my + d, N_DEV)
            pl.semaphore_signal(bar, inc=1, device_id=(peer,),
                                device_id_type=pl.DeviceIdType.MESH)
        pl.semaphore_wait(bar, N_DEV - 1)

        xdmas[0].wait()
        a8[:, pl.ds(my * K_BLK, K_BLK)] = xv8[0].astype(FP8)
        rdmas = []
        for d in range(1, N_DEV):
            dst = lax.rem(my + d, N_DEV)
            xdmas[d].wait()
            xq8[d] = xv8[d].astype(FP8)
            r = pltpu.make_async_remote_copy(
                src_ref=xq8.at[d],
                dst_ref=a8.at[:, pl.ds(my * K_BLK, K_BLK)],
                send_sem=send_sems.at[d - 1],
                recv_sem=recv_sems.at[my],
                device_id=(dst,),
                device_id_type=pl.DeviceIdType.MESH,
            )
            r.start()
            rdmas.append(r)

        for d in range(1, N_DEV):
            k = lax.rem(my + d, N_DEV)
            pltpu.make_async_remote_copy(
                src_ref=xq8.at[d],
                dst_ref=a8.at[:, pl.ds(k * K_BLK, K_BLK)],
                send_sem=send_sems.at[0],
                recv_sem=recv_sems.at[k],
                device_id=(my,),
                device_id_type=pl.DeviceIdType.MESH,
            ).wait_recv()

        s = sx_ref[0] * sw_ref[0]

        for t in range(N_STRIPES):
            slot = t % S
            oslot = t % 2
            pending[slot].wait()
            b = wbuf[slot].astype(FP8)
            prod = lax.dot_general(
                a8[...], b, (((1,), (0,)), ((), ())),
                preferred_element_type=jnp.float32,
            )
            if t + S < N_STRIPES:
                nd = w_dma(t + S, slot)
                nd.start()
                pending[slot] = nd
            y = prod * s
            res = y / (1.0 + jnp.exp(-jnp.clip(y, -60.0, 60.0)))
            if t >= 2:
                pltpu.make_async_copy(
                    obuf.at[oslot],
                    out_hbm.at[:, pl.ds((t - 2) * NB, NB)],
                    out_sems.at[oslot],
                ).wait()
            obuf[oslot] = res
            pltpu.make_async_copy(
                obuf.at[oslot],
                out_hbm.at[:, pl.ds(t * NB, NB)],
                out_sems.at[oslot],
            ).start()

        for t in (N_STRIPES - 2, N_STRIPES - 1):
            pltpu.make_async_copy(
                obuf.at[t % 2],
                out_hbm.at[:, pl.ds(t * NB, NB)],
                out_sems.at[t % 2],
            ).wait()
        for r in rdmas:
            r.wait_send()

    return pl.pallas_call(
        body,
        out_shape=jax.ShapeDtypeStruct((M_BLK, N_TOT), jnp.float32),
        in_specs=[
            pl.BlockSpec(memory_space=pl.ANY),
            pl.BlockSpec(memory_space=pl.ANY),
            pl.BlockSpec(memory_space=pltpu.SMEM),
            pl.BlockSpec(memory_space=pltpu.SMEM),
        ],
        out_specs=pl.BlockSpec(memory_space=pl.ANY),
        scratch_shapes=[
            pltpu.VMEM((N_DEV, M_BLK, K_BLK), jnp.float32),
            pltpu.VMEM((N_DEV, M_BLK, K_BLK), FP8),
            pltpu.VMEM((M_BLK, K_TOT), FP8),
            pltpu.VMEM((S, K_TOT, NB), jnp.float32),
            pltpu.VMEM((2, M_BLK, NB), jnp.float32),
            pltpu.SemaphoreType.DMA((N_DEV,)),
            pltpu.SemaphoreType.DMA((N_DEV - 1,)),
            pltpu.SemaphoreType.DMA((N_DEV,)),
            pltpu.SemaphoreType.DMA((S,)),
            pltpu.SemaphoreType.DMA((2,)),
        ],
        compiler_params=pltpu.CompilerParams(
            collective_id=0, vmem_limit_bytes=100 * 1024 * 1024
        ),
    )(x, w_mat, scale_x, scale_w)
